# baseline (speedup 1.0000x reference)
import functools

import jax
import jax.numpy as jnp
from jax import lax
from jax.experimental import pallas as pl
from jax.experimental.pallas import tpu as pltpu
from jax.experimental.pallas import tpu_sc as plsc

VOCAB = 1000000
B = 4096
L = 200
D = 32
NUM_LABELS = 10

NC = 2
NS = 16
NW = NC * NS
RPW = B // NW
NBUF = 4
LANES = 16

L_SPLIT = 128
L_REST = L - L_SPLIT


def _sc_gather_sum(ids_flat, emb):
    mesh = plsc.VectorSubcoreMesh(
        core_axis_name="c", subcore_axis_name="s",
        num_cores=NC, num_subcores=NS)

    @functools.partial(
        pl.kernel,
        out_type=jax.ShapeDtypeStruct((B, D), jnp.float32),
        mesh=mesh,
        compiler_params=pltpu.CompilerParams(use_tc_tiling_on_sc=False),
        scratch_types=dict(
            idx_v=pltpu.VMEM((RPW * L,), jnp.int32),
            rows_v=[pltpu.VMEM((L, D), jnp.float32) for _ in range(NBUF)],
            out_v=pltpu.VMEM((RPW, D), jnp.float32),
            sems=[pltpu.SemaphoreType.DMA for _ in range(NBUF)],
        ),
    )
    def k(ids_hbm, emb_hbm, out_hbm, idx_v, rows_v, out_v, sems):
        wid = lax.axis_index("s") * NC + lax.axis_index("c")
        base = wid * RPW

        pltpu.sync_copy(ids_hbm.at[pl.ds(base * L, RPW * L)], idx_v)

        def fetch(r, buf, sem):
            pltpu.async_copy(
                emb_hbm.at[idx_v.at[pl.ds(r * L, L_SPLIT)]],
                buf.at[pl.ds(0, L_SPLIT)], sem)
            pltpu.async_copy(
                emb_hbm.at[idx_v.at[pl.ds(r * L + L_SPLIT, L_REST)]],
                buf.at[pl.ds(L_SPLIT, L_REST)], sem)

        def drain(r, buf, sem):
            pltpu.make_async_copy(
                emb_hbm.at[idx_v.at[pl.ds(r * L, L_SPLIT)]],
                buf.at[pl.ds(0, L_SPLIT)], sem).wait()
            pltpu.make_async_copy(
                emb_hbm.at[idx_v.at[pl.ds(r * L + L_SPLIT, L_REST)]],
                buf.at[pl.ds(L_SPLIT, L_REST)], sem).wait()

        for j in range(NBUF):
            fetch(j, rows_v[j], sems[j])

        zero = jnp.zeros((LANES,), jnp.float32)

        def outer(i, carry):
            r0 = i * NBUF
            for j in range(NBUF):
                r = r0 + j
                drain(r, rows_v[j], sems[j])

                def red(l, acc):
                    a0, a1 = acc
                    a0 = a0 + rows_v[j][l, pl.ds(0, LANES)]
                    a1 = a1 + rows_v[j][l, pl.ds(LANES, LANES)]
                    return (a0, a1)

                a0, a1 = lax.fori_loop(0, L, red, (zero, zero), unroll=8)
                out_v[r, pl.ds(0, LANES)] = a0
                out_v[r, pl.ds(LANES, LANES)] = a1

                @pl.when(r + NBUF < RPW)
                def _():
                    fetch(r + NBUF, rows_v[j], sems[j])
            return carry

        lax.fori_loop(0, RPW // NBUF, outer, 0)

        pltpu.sync_copy(out_v, out_hbm.at[pl.ds(base, RPW)])

    return k(ids_flat, emb)


TBLK = 8192


def _tc_relayout(emb_t):
    def body(in_ref, o_ref):
        o_ref[...] = lax.dot_general(
            in_ref[...], jnp.eye(D, dtype=jnp.float32),
            ((( 0,), (0,)), ((), ())),
            precision=lax.Precision.HIGHEST,
            preferred_element_type=jnp.float32)

    grid = (VOCAB + TBLK - 1) // TBLK
    return pl.pallas_call(
        body,
        grid=(grid,),
        in_specs=[pl.BlockSpec((D, TBLK), lambda i: (0, i))],
        out_specs=pl.BlockSpec((TBLK, D), lambda i: (i, 0)),
        out_shape=jax.ShapeDtypeStruct((VOCAB, D), jnp.float32),
        compiler_params=pltpu.CompilerParams(
            dimension_semantics=("parallel",)),
    )(emb_t)


def _tc_head(sums, lengths2d, wt, b2):
    def body(s_ref, len_ref, wt_ref, b_ref, o_ref):
        den = jnp.maximum(len_ref[...].astype(jnp.float32), 1.0)
        mean = s_ref[...] / den
        o_ref[...] = (
            jnp.dot(mean, wt_ref[...], preferred_element_type=jnp.float32)
            + b_ref[...])

    return pl.pallas_call(
        body,
        out_shape=jax.ShapeDtypeStruct((B, NUM_LABELS), jnp.float32),
    )(sums, lengths2d, wt, b2)


def kernel(ids, lengths, emb, W, b):
    emb_rm = _tc_relayout(emb.T)
    sums = _sc_gather_sum(ids.reshape(B * L), emb_rm)
    return _tc_head(sums, lengths.reshape(B, 1), W.T, b.reshape(1, NUM_LABELS))


# speedup vs baseline: 1.5166x; 1.5166x over previous
import functools

import jax
import jax.numpy as jnp
from jax import lax
from jax.experimental import pallas as pl
from jax.experimental.pallas import tpu as pltpu
from jax.experimental.pallas import tpu_sc as plsc

VOCAB = 1000000
B = 4096
L = 200
D = 32
NUM_LABELS = 10

NC = 2
NS = 16
NW = NC * NS
RPW = B // NW
NBUF = 4
LANES = 16

L_SPLIT = 128
L_REST = L - L_SPLIT


def _sc_gather_sum(ids_flat, emb):
    mesh = plsc.VectorSubcoreMesh(
        core_axis_name="c", subcore_axis_name="s",
        num_cores=NC, num_subcores=NS)

    @functools.partial(
        pl.kernel,
        out_type=jax.ShapeDtypeStruct((B, D), jnp.float32),
        mesh=mesh,
        compiler_params=pltpu.CompilerParams(use_tc_tiling_on_sc=False),
        scratch_types=dict(
            idx_v=pltpu.VMEM((RPW * L,), jnp.int32),
            rows_v=[pltpu.VMEM((L, D), jnp.float32) for _ in range(NBUF)],
            out_v=pltpu.VMEM((RPW, D), jnp.float32),
            sems=[pltpu.SemaphoreType.DMA for _ in range(NBUF)],
        ),
    )
    def k(ids_hbm, emb_hbm, out_hbm, idx_v, rows_v, out_v, sems):
        wid = lax.axis_index("s") * NC + lax.axis_index("c")
        base = wid * RPW

        pltpu.sync_copy(ids_hbm.at[pl.ds(base * L, RPW * L)], idx_v)

        def fetch(r, buf, sem):
            pltpu.async_copy(
                emb_hbm.at[idx_v.at[pl.ds(r * L, L_SPLIT)]],
                buf.at[pl.ds(0, L_SPLIT)], sem)
            pltpu.async_copy(
                emb_hbm.at[idx_v.at[pl.ds(r * L + L_SPLIT, L_REST)]],
                buf.at[pl.ds(L_SPLIT, L_REST)], sem)

        def drain(r, buf, sem):
            pltpu.make_async_copy(
                emb_hbm.at[idx_v.at[pl.ds(r * L, L_SPLIT)]],
                buf.at[pl.ds(0, L_SPLIT)], sem).wait()
            pltpu.make_async_copy(
                emb_hbm.at[idx_v.at[pl.ds(r * L + L_SPLIT, L_REST)]],
                buf.at[pl.ds(L_SPLIT, L_REST)], sem).wait()

        for j in range(NBUF):
            fetch(j, rows_v[j], sems[j])

        zero = jnp.zeros((LANES,), jnp.float32)

        def outer(i, carry):
            r0 = i * NBUF
            for j in range(NBUF):
                r = r0 + j
                drain(r, rows_v[j], sems[j])

                def red(l, acc):
                    a0, a1 = acc
                    a0 = a0 + rows_v[j][l, pl.ds(0, LANES)]
                    a1 = a1 + rows_v[j][l, pl.ds(LANES, LANES)]
                    return (a0, a1)

                a0, a1 = lax.fori_loop(0, L, red, (zero, zero), unroll=8)
                out_v[r, pl.ds(0, LANES)] = a0
                out_v[r, pl.ds(LANES, LANES)] = a1

                @pl.when(r + NBUF < RPW)
                def _():
                    fetch(r + NBUF, rows_v[j], sems[j])
            return carry

        lax.fori_loop(0, RPW // NBUF, outer, 0)

        pltpu.sync_copy(out_v, out_hbm.at[pl.ds(base, RPW)])

    return k(ids_flat, emb)


TBLK = 8192


def _tc_relayout(emb_t):
    def body(in_ref, o_ref):
        o_ref[...] = lax.dot_general(
            in_ref[...], jnp.eye(D, dtype=jnp.float32),
            ((( 0,), (0,)), ((), ())),
            precision=lax.Precision.HIGHEST,
            preferred_element_type=jnp.float32)

    grid = (VOCAB + TBLK - 1) // TBLK
    return pl.pallas_call(
        body,
        grid=(grid,),
        in_specs=[pl.BlockSpec((D, TBLK), lambda i: (0, i))],
        out_specs=pl.BlockSpec((TBLK, D), lambda i: (i, 0)),
        out_shape=jax.ShapeDtypeStruct((VOCAB, D), jnp.float32),
        compiler_params=pltpu.CompilerParams(
            dimension_semantics=("parallel",)),
    )(emb_t)


def _tc_head(sums, lengths2d, wt, b2):
    def body(s_ref, len_ref, wt_ref, b_ref, o_ref):
        den = jnp.maximum(len_ref[...].astype(jnp.float32), 1.0)
        mean = s_ref[...] / den
        o_ref[...] = (
            jnp.dot(mean, wt_ref[...], preferred_element_type=jnp.float32)
            + b_ref[...])

    return pl.pallas_call(
        body,
        out_shape=jax.ShapeDtypeStruct((B, NUM_LABELS), jnp.float32),
    )(sums, lengths2d, wt, b2)


def kernel(ids, lengths, emb, W, b):
    sums = _sc_gather_sum(ids.reshape(B * L), emb)
    return _tc_head(sums, lengths.reshape(B, 1), W.T, b.reshape(1, NUM_LABELS))


# speedup vs baseline: 1.5362x; 1.0130x over previous
import functools

import jax
import jax.numpy as jnp
from jax import lax
from jax.experimental import pallas as pl
from jax.experimental.pallas import tpu as pltpu
from jax.experimental.pallas import tpu_sc as plsc

VOCAB = 1000000
B = 4096
L = 200
D = 32
NUM_LABELS = 10

NC = 2
NS = 16
NW = NC * NS
RPW = B // NW
NBUF = 8
LANES = 16

L_SPLIT = 128
L_REST = L - L_SPLIT


def _sc_gather_sum(ids_flat, emb):
    mesh = plsc.VectorSubcoreMesh(
        core_axis_name="c", subcore_axis_name="s",
        num_cores=NC, num_subcores=NS)

    @functools.partial(
        pl.kernel,
        out_type=jax.ShapeDtypeStruct((B, D), jnp.float32),
        mesh=mesh,
        compiler_params=pltpu.CompilerParams(use_tc_tiling_on_sc=False),
        scratch_types=dict(
            idx_v=pltpu.VMEM((RPW * L,), jnp.int32),
            rows_v=[pltpu.VMEM((L, D), jnp.float32) for _ in range(NBUF)],
            out_v=pltpu.VMEM((RPW, D), jnp.float32),
            sems=[pltpu.SemaphoreType.DMA for _ in range(NBUF)],
        ),
    )
    def k(ids_hbm, emb_hbm, out_hbm, idx_v, rows_v, out_v, sems):
        wid = lax.axis_index("s") * NC + lax.axis_index("c")
        base = wid * RPW

        pltpu.sync_copy(ids_hbm.at[pl.ds(base * L, RPW * L)], idx_v)

        def fetch(r, buf, sem):
            pltpu.async_copy(
                emb_hbm.at[idx_v.at[pl.ds(r * L, L_SPLIT)]],
                buf.at[pl.ds(0, L_SPLIT)], sem)
            pltpu.async_copy(
                emb_hbm.at[idx_v.at[pl.ds(r * L + L_SPLIT, L_REST)]],
                buf.at[pl.ds(L_SPLIT, L_REST)], sem)

        def drain(r, buf, sem):
            pltpu.make_async_copy(
                emb_hbm.at[idx_v.at[pl.ds(r * L, L_SPLIT)]],
                buf.at[pl.ds(0, L_SPLIT)], sem).wait()
            pltpu.make_async_copy(
                emb_hbm.at[idx_v.at[pl.ds(r * L + L_SPLIT, L_REST)]],
                buf.at[pl.ds(L_SPLIT, L_REST)], sem).wait()

        for j in range(NBUF):
            fetch(j, rows_v[j], sems[j])

        zero = jnp.zeros((LANES,), jnp.float32)

        def outer(i, carry):
            r0 = i * NBUF
            for j in range(NBUF):
                r = r0 + j
                drain(r, rows_v[j], sems[j])

                def red(l, acc):
                    a0, a1 = acc
                    a0 = a0 + rows_v[j][l, pl.ds(0, LANES)]
                    a1 = a1 + rows_v[j][l, pl.ds(LANES, LANES)]
                    return (a0, a1)

                a0, a1 = lax.fori_loop(0, L, red, (zero, zero), unroll=8)
                out_v[r, pl.ds(0, LANES)] = a0
                out_v[r, pl.ds(LANES, LANES)] = a1

                @pl.when(r + NBUF < RPW)
                def _():
                    fetch(r + NBUF, rows_v[j], sems[j])
            return carry

        lax.fori_loop(0, RPW // NBUF, outer, 0)

        pltpu.sync_copy(out_v, out_hbm.at[pl.ds(base, RPW)])

    return k(ids_flat, emb)


TBLK = 8192


def _tc_relayout(emb_t):
    def body(in_ref, o_ref):
        o_ref[...] = lax.dot_general(
            in_ref[...], jnp.eye(D, dtype=jnp.float32),
            ((( 0,), (0,)), ((), ())),
            precision=lax.Precision.HIGHEST,
            preferred_element_type=jnp.float32)

    grid = (VOCAB + TBLK - 1) // TBLK
    return pl.pallas_call(
        body,
        grid=(grid,),
        in_specs=[pl.BlockSpec((D, TBLK), lambda i: (0, i))],
        out_specs=pl.BlockSpec((TBLK, D), lambda i: (i, 0)),
        out_shape=jax.ShapeDtypeStruct((VOCAB, D), jnp.float32),
        compiler_params=pltpu.CompilerParams(
            dimension_semantics=("parallel",)),
    )(emb_t)


def _tc_head(sums, lengths2d, wt, b2):
    def body(s_ref, len_ref, wt_ref, b_ref, o_ref):
        den = jnp.maximum(len_ref[...].astype(jnp.float32), 1.0)
        mean = s_ref[...] / den
        o_ref[...] = (
            jnp.dot(mean, wt_ref[...], preferred_element_type=jnp.float32)
            + b_ref[...])

    return pl.pallas_call(
        body,
        out_shape=jax.ShapeDtypeStruct((B, NUM_LABELS), jnp.float32),
    )(sums, lengths2d, wt, b2)


def kernel(ids, lengths, emb, W, b):
    sums = _sc_gather_sum(ids.reshape(B * L), emb)
    return _tc_head(sums, lengths.reshape(B, 1), W.T, b.reshape(1, NUM_LABELS))


# speedup vs baseline: 1.5373x; 1.0007x over previous
import functools

import jax
import jax.numpy as jnp
from jax import lax
from jax.experimental import pallas as pl
from jax.experimental.pallas import tpu as pltpu
from jax.experimental.pallas import tpu_sc as plsc

VOCAB = 1000000
B = 4096
L = 200
D = 32
NUM_LABELS = 10

NC = 2
NS = 16
NW = NC * NS
RPW = B // NW
NBUF = 8
LANES = 16

L_SPLIT = 128
L_REST = L - L_SPLIT


def _sc_gather_sum(ids_flat, emb):
    mesh = plsc.VectorSubcoreMesh(
        core_axis_name="c", subcore_axis_name="s",
        num_cores=NC, num_subcores=NS)

    @functools.partial(
        pl.kernel,
        out_type=jax.ShapeDtypeStruct((B, D), jnp.float32),
        mesh=mesh,
        compiler_params=pltpu.CompilerParams(use_tc_tiling_on_sc=False),
        scratch_types=dict(
            idx_v=pltpu.VMEM((RPW * L,), jnp.int32),
            rows_v=[pltpu.VMEM((L, D), jnp.float32) for _ in range(NBUF)],
            out_v=pltpu.VMEM((RPW, D), jnp.float32),
            sems=[pltpu.SemaphoreType.DMA for _ in range(NBUF)],
        ),
    )
    def k(ids_hbm, emb_hbm, out_hbm, idx_v, rows_v, out_v, sems):
        wid = lax.axis_index("s") * NC + lax.axis_index("c")
        base = wid * RPW

        pltpu.sync_copy(ids_hbm.at[pl.ds(base * L, RPW * L)], idx_v)

        def fetch(r, buf, sem):
            pltpu.async_copy(
                emb_hbm.at[idx_v.at[pl.ds(r * L, L_SPLIT)]],
                buf.at[pl.ds(0, L_SPLIT)], sem)
            pltpu.async_copy(
                emb_hbm.at[idx_v.at[pl.ds(r * L + L_SPLIT, L_REST)]],
                buf.at[pl.ds(L_SPLIT, L_REST)], sem)

        def drain(r, buf, sem):
            pltpu.make_async_copy(
                emb_hbm.at[idx_v.at[pl.ds(r * L, L_SPLIT)]],
                buf.at[pl.ds(0, L_SPLIT)], sem).wait()
            pltpu.make_async_copy(
                emb_hbm.at[idx_v.at[pl.ds(r * L + L_SPLIT, L_REST)]],
                buf.at[pl.ds(L_SPLIT, L_REST)], sem).wait()

        for j in range(NBUF):
            fetch(j, rows_v[j], sems[j])

        zero = jnp.zeros((LANES,), jnp.float32)

        def outer(i, carry):
            r0 = i * NBUF
            for j in range(NBUF):
                r = r0 + j
                drain(r, rows_v[j], sems[j])

                def red(l, acc):
                    a0, a1 = acc
                    a0 = a0 + rows_v[j][l, pl.ds(0, LANES)]
                    a1 = a1 + rows_v[j][l, pl.ds(LANES, LANES)]
                    return (a0, a1)

                a0, a1 = lax.fori_loop(0, L, red, (zero, zero), unroll=8)
                out_v[r, pl.ds(0, LANES)] = a0
                out_v[r, pl.ds(LANES, LANES)] = a1

                @pl.when(r + NBUF < RPW)
                def _():
                    fetch(r + NBUF, rows_v[j], sems[j])
            return carry

        lax.fori_loop(0, RPW // NBUF, outer, 0)

        pltpu.sync_copy(out_v, out_hbm.at[pl.ds(base, RPW)])

    return k(ids_flat, emb)


def _tc_head(sums, lengths2d, wt, b2):
    def body(s_ref, len_ref, wt_ref, b_ref, o_ref):
        den = jnp.maximum(len_ref[...].astype(jnp.float32), 1.0)
        mean = s_ref[...] / den
        o_ref[...] = (
            jnp.dot(mean, wt_ref[...], preferred_element_type=jnp.float32)
            + b_ref[...])

    return pl.pallas_call(
        body,
        out_shape=jax.ShapeDtypeStruct((B, NUM_LABELS), jnp.float32),
    )(sums, lengths2d, wt, b2)


def kernel(ids, lengths, emb, W, b):
    sums = _sc_gather_sum(ids.reshape(B * L), emb)
    return _tc_head(sums, lengths.reshape(B, 1), W.T, b.reshape(1, NUM_LABELS))
